# baseline (speedup 1.0000x reference)
import functools

import jax
import jax.numpy as jnp
from jax import lax
from jax.experimental import pallas as pl
from jax.experimental.pallas import tpu as pltpu
from jax.experimental.pallas import tpu_sc as plsc

VOCAB = 100000
HIDDEN = 128
BATCH = 1024


_NC, _NS = 2, 16
_NW = _NC * _NS
_B_PER_W = BATCH // _NW


@functools.cache
def _make_sc_gather():
    mesh = plsc.VectorSubcoreMesh(core_axis_name="c", subcore_axis_name="s")

    half = _B_PER_W // 2

    @functools.partial(
        pl.kernel,
        mesh=mesh,
        out_type=jax.ShapeDtypeStruct((BATCH, HIDDEN), jnp.float32),
        scratch_types=[
            pltpu.VMEM((_B_PER_W,), jnp.int32),
            pltpu.VMEM((half, HIDDEN), jnp.float32),
            pltpu.VMEM((half, HIDDEN), jnp.float32),
            pltpu.SemaphoreType.DMA,
            pltpu.SemaphoreType.DMA,
            pltpu.SemaphoreType.DMA,
            pltpu.SemaphoreType.DMA,
        ],
    )
    def sc_gather(table_hbm, idx_hbm, out_hbm, idx_v, rows0, rows1,
                  g0, g1, o0, o1):
        wid = lax.axis_index("s") * _NC + lax.axis_index("c")
        base = wid * _B_PER_W
        pltpu.sync_copy(idx_hbm.at[pl.ds(base, _B_PER_W)], idx_v)
        c0 = pltpu.async_copy(table_hbm.at[idx_v.at[pl.ds(0, half)]], rows0, g0)
        c1 = pltpu.async_copy(table_hbm.at[idx_v.at[pl.ds(half, half)]], rows1, g1)
        c0.wait()
        w0 = pltpu.async_copy(rows0, out_hbm.at[pl.ds(base, half)], o0)
        c1.wait()
        w1 = pltpu.async_copy(rows1, out_hbm.at[pl.ds(base + half, half)], o1)
        w0.wait()
        w1.wait()

    return sc_gather


_TILE_V = 4096
_NBLK = pl.cdiv(VOCAB, _TILE_V)


def _proj_body(hidden_ref, w_ref, b_ref, out_ref):
    acc = lax.dot_general(
        w_ref[...], hidden_ref[...],
        dimension_numbers=(((1,), (1,)), ((), ())),
        preferred_element_type=jnp.float32,
    )
    out_ref[...] = acc + jnp.transpose(b_ref[...][None, :])


def _projection(hidden, lm_w, lm_b):
    return pl.pallas_call(
        _proj_body,
        grid=(_NBLK,),
        in_specs=[
            pl.BlockSpec((BATCH, HIDDEN), lambda j: (0, 0)),
            pl.BlockSpec((_TILE_V, HIDDEN), lambda j: (j, 0)),
            pl.BlockSpec((_TILE_V,), lambda j: (j,)),
        ],
        out_specs=pl.BlockSpec((_TILE_V, BATCH), lambda j: (j, 0)),
        out_shape=jax.ShapeDtypeStruct((VOCAB, BATCH), jnp.float32),
    )(hidden, lm_w, lm_b)


def kernel(input_ids, embed_table, lm_w, lm_b):
    hidden = _make_sc_gather()(embed_table, input_ids)
    logits_t = _projection(hidden, lm_w, lm_b)
    return logits_t.T


# speedup vs baseline: 1.0027x; 1.0027x over previous
import functools

import jax
import jax.numpy as jnp
from jax import lax
from jax.experimental import pallas as pl
from jax.experimental.pallas import tpu as pltpu
from jax.experimental.pallas import tpu_sc as plsc

VOCAB = 100000
HIDDEN = 128
BATCH = 1024


_NC, _NS = 2, 16
_NW = _NC * _NS
_B_PER_W = BATCH // _NW


@functools.cache
def _make_sc_gather():
    mesh = plsc.VectorSubcoreMesh(core_axis_name="c", subcore_axis_name="s")

    half = _B_PER_W // 2

    @functools.partial(
        pl.kernel,
        mesh=mesh,
        out_type=jax.ShapeDtypeStruct((BATCH, HIDDEN), jnp.float32),
        scratch_types=[
            pltpu.VMEM((_B_PER_W,), jnp.int32),
            pltpu.VMEM((half, HIDDEN), jnp.float32),
            pltpu.VMEM((half, HIDDEN), jnp.float32),
            pltpu.SemaphoreType.DMA,
            pltpu.SemaphoreType.DMA,
            pltpu.SemaphoreType.DMA,
            pltpu.SemaphoreType.DMA,
        ],
    )
    def sc_gather(table_hbm, idx_hbm, out_hbm, idx_v, rows0, rows1,
                  g0, g1, o0, o1):
        wid = lax.axis_index("s") * _NC + lax.axis_index("c")
        base = wid * _B_PER_W
        pltpu.sync_copy(idx_hbm.at[pl.ds(base, _B_PER_W)], idx_v)
        c0 = pltpu.async_copy(table_hbm.at[idx_v.at[pl.ds(0, half)]], rows0, g0)
        c1 = pltpu.async_copy(table_hbm.at[idx_v.at[pl.ds(half, half)]], rows1, g1)
        c0.wait()
        w0 = pltpu.async_copy(rows0, out_hbm.at[pl.ds(base, half)], o0)
        c1.wait()
        w1 = pltpu.async_copy(rows1, out_hbm.at[pl.ds(base + half, half)], o1)
        w0.wait()
        w1.wait()

    return sc_gather


_TILE_V = 5120
_NBLK = pl.cdiv(VOCAB, _TILE_V)


def _proj_body(hidden_ref, w_ref, b_ref, out_ref):
    acc = lax.dot_general(
        w_ref[...], hidden_ref[...],
        dimension_numbers=(((1,), (1,)), ((), ())),
        preferred_element_type=jnp.float32,
    )
    out_ref[...] = acc + jnp.transpose(b_ref[...][None, :])


def _projection(hidden, lm_w, lm_b):
    return pl.pallas_call(
        _proj_body,
        grid=(_NBLK,),
        in_specs=[
            pl.BlockSpec((BATCH, HIDDEN), lambda j: (0, 0)),
            pl.BlockSpec((_TILE_V, HIDDEN), lambda j: (j, 0)),
            pl.BlockSpec((_TILE_V,), lambda j: (j,)),
        ],
        out_specs=pl.BlockSpec((_TILE_V, BATCH), lambda j: (j, 0)),
        out_shape=jax.ShapeDtypeStruct((VOCAB, BATCH), jnp.float32),
    )(hidden, lm_w, lm_b)


def kernel(input_ids, embed_table, lm_w, lm_b):
    hidden = _make_sc_gather()(embed_table, input_ids)
    logits_t = _projection(hidden, lm_w, lm_b)
    return logits_t.T
